# baseline (speedup 1.0000x reference)
import functools

import jax
import jax.numpy as jnp
from jax import lax
from jax.experimental import pallas as pl
from jax.experimental.pallas import tpu as pltpu

D = 1024
F = 2816
H = 16
DH = 64
EPS = 1e-6
CAP = 0.5
TB = 256
FB = 1408


def _rms_matmul_kern(x_ref, s_ref, w_ref, o_ref):
    x = x_ref[:]
    v = jnp.mean(jnp.square(x), axis=-1, keepdims=True)
    xn = (x * lax.rsqrt(v + EPS)) * s_ref[:]
    o_ref[:] = jnp.dot(xn, w_ref[:], preferred_element_type=jnp.float32)


def _rms_matmul(x, s, w):
    T, _ = x.shape
    N = w.shape[1]
    return pl.pallas_call(
        _rms_matmul_kern,
        grid=(T // TB,),
        in_specs=[
            pl.BlockSpec((TB, D), lambda i: (i, 0)),
            pl.BlockSpec((1, D), lambda i: (0, 0)),
            pl.BlockSpec((D, N), lambda i: (0, 0)),
        ],
        out_specs=pl.BlockSpec((TB, N), lambda i: (i, 0)),
        out_shape=jax.ShapeDtypeStruct((T, N), jnp.float32),
    )(x, s, w)


def _attn_kern(q_ref, k_ref, v_ref, o_ref):
    qb = pl.program_id(1)
    q = q_ref[0]

    def body(kb, carry):
        m, l, acc = carry
        start = pl.multiple_of(kb * TB, TB)
        kblk = k_ref[0, pl.ds(start, TB), :]
        vblk = v_ref[0, pl.ds(start, TB), :]
        s = lax.dot_general(q, kblk, (((1,), (1,)), ((), ())),
                            preferred_element_type=jnp.float32) * 0.125
        row = qb * TB + lax.broadcasted_iota(jnp.int32, (TB, TB), 0)
        col = start + lax.broadcasted_iota(jnp.int32, (TB, TB), 1)
        s = jnp.where(row >= col, s, -1e9)
        m_new = jnp.maximum(m, jnp.max(s, axis=1, keepdims=True))
        alpha = jnp.exp(m - m_new)
        p = jnp.exp(s - m_new)
        l_new = l * alpha + jnp.sum(p, axis=1, keepdims=True)
        acc_new = acc * alpha + jnp.dot(p, vblk,
                                        preferred_element_type=jnp.float32)
        return m_new, l_new, acc_new

    m0 = jnp.full((TB, 1), -1e30, jnp.float32)
    l0 = jnp.zeros((TB, 1), jnp.float32)
    acc0 = jnp.zeros((TB, DH), jnp.float32)
    m, l, acc = lax.fori_loop(0, qb + 1, body, (m0, l0, acc0))
    o_ref[0] = acc / l


def _attention(q, k, v):
    T = q.shape[1]
    return pl.pallas_call(
        _attn_kern,
        grid=(H, T // TB),
        in_specs=[
            pl.BlockSpec((1, TB, DH), lambda h, i: (h, i, 0)),
            pl.BlockSpec((1, T, DH), lambda h, i: (h, 0, 0)),
            pl.BlockSpec((1, T, DH), lambda h, i: (h, 0, 0)),
        ],
        out_specs=pl.BlockSpec((1, TB, DH), lambda h, i: (h, i, 0)),
        out_shape=jax.ShapeDtypeStruct((H, T, DH), jnp.float32),
    )(q, k, v)


def _matmul_add_kern(a_ref, w_ref, r_ref, o_ref):
    o_ref[:] = r_ref[:] + jnp.dot(a_ref[:], w_ref[:],
                                  preferred_element_type=jnp.float32)


def _matmul_add(a, w, r):
    T = a.shape[0]
    N = w.shape[1]
    return pl.pallas_call(
        _matmul_add_kern,
        grid=(T // TB,),
        in_specs=[
            pl.BlockSpec((TB, D), lambda i: (i, 0)),
            pl.BlockSpec((D, N), lambda i: (0, 0)),
            pl.BlockSpec((TB, N), lambda i: (i, 0)),
        ],
        out_specs=pl.BlockSpec((TB, N), lambda i: (i, 0)),
        out_shape=jax.ShapeDtypeStruct((T, N), jnp.float32),
    )(a, w, r)


def _mlp_kern(x_ref, s_ref, wg_ref, wu_ref, wd_ref, o_ref):
    fb = pl.program_id(1)
    x = x_ref[:]
    v = jnp.mean(jnp.square(x), axis=-1, keepdims=True)
    xn = (x * lax.rsqrt(v + EPS)) * s_ref[:]
    g = jnp.dot(xn, wg_ref[:], preferred_element_type=jnp.float32)
    u = jnp.dot(xn, wu_ref[:], preferred_element_type=jnp.float32)
    t = (g * jax.nn.sigmoid(g)) * u
    part = jnp.dot(t, wd_ref[:], preferred_element_type=jnp.float32)

    @pl.when(fb == 0)
    def _():
        o_ref[:] = x + part

    @pl.when(fb != 0)
    def _():
        o_ref[:] = o_ref[:] + part


def _mlp_residual(x, s, wg, wu, wd):
    T = x.shape[0]
    return pl.pallas_call(
        _mlp_kern,
        grid=(T // TB, F // FB),
        in_specs=[
            pl.BlockSpec((TB, D), lambda i, j: (i, 0)),
            pl.BlockSpec((1, D), lambda i, j: (0, 0)),
            pl.BlockSpec((D, FB), lambda i, j: (0, j)),
            pl.BlockSpec((D, FB), lambda i, j: (0, j)),
            pl.BlockSpec((FB, D), lambda i, j: (j, 0)),
        ],
        out_specs=pl.BlockSpec((TB, D), lambda i, j: (i, 0)),
        out_shape=jax.ShapeDtypeStruct((T, D), jnp.float32),
    )(x, s, wg, wu, wd)


def _score_kern(prev_ref, s_ref, wg_ref, wu_ref, wd_ref, proc_ref, orig_ref,
                bce_ref, bcu_ref, g_ref, pred_acc):
    fb = pl.program_id(1)
    nfb = pl.num_programs(1)
    x = prev_ref[:]
    v = jnp.mean(jnp.square(x), axis=-1, keepdims=True)
    xn = (x * lax.rsqrt(v + EPS)) * s_ref[:]
    g = jnp.dot(xn, wg_ref[:], preferred_element_type=jnp.float32)
    u = jnp.dot(xn, wu_ref[:], preferred_element_type=jnp.float32)
    t = (g * jax.nn.sigmoid(g)) * u
    part = jnp.dot(t, wd_ref[:], preferred_element_type=jnp.float32)

    @pl.when(fb == 0)
    def _():
        pred_acc[:] = part

    @pl.when(fb != 0)
    def _():
        pred_acc[:] = pred_acc[:] + part

    @pl.when(fb == nfb - 1)
    def _():
        pred = pred_acc[:]
        res = proc_ref[:] - orig_ref[:]
        d_st = jnp.sum(res * res, axis=-1, keepdims=True) * (1.0 / D)
        e = res - pred
        d_ch = jnp.sum(e * e, axis=-1, keepdims=True) * (1.0 / D)
        logit = bce_ref[0, 0] * d_st - bcu_ref[0, 0] * d_ch
        g_ref[:] = jax.nn.sigmoid(logit)


def _tn_scores(prev, s, wg, wu, wd, proc, orig, bce, bcu):
    T = prev.shape[0]
    return pl.pallas_call(
        _score_kern,
        grid=(T // TB, F // FB),
        in_specs=[
            pl.BlockSpec((TB, D), lambda i, j: (i, 0)),
            pl.BlockSpec((1, D), lambda i, j: (0, 0)),
            pl.BlockSpec((D, FB), lambda i, j: (0, j)),
            pl.BlockSpec((D, FB), lambda i, j: (0, j)),
            pl.BlockSpec((FB, D), lambda i, j: (j, 0)),
            pl.BlockSpec((TB, D), lambda i, j: (i, 0)),
            pl.BlockSpec((TB, D), lambda i, j: (i, 0)),
            pl.BlockSpec((1, 1), lambda i, j: (0, 0)),
            pl.BlockSpec((1, 1), lambda i, j: (0, 0)),
        ],
        out_specs=pl.BlockSpec((TB, 1), lambda i, j: (i, 0)),
        out_shape=jax.ShapeDtypeStruct((T, 1), jnp.float32),
        scratch_shapes=[pltpu.VMEM((TB, D), jnp.float32)],
    )(prev, s, wg, wu, wd, proc, orig, bce, bcu)


def _gate_combine_kern(grow_ref, gt_ref, orig_ref, proc_ref, o_ref, *, k, T):
    i = pl.program_id(0)
    g_row = grow_ref[:]
    g_t = gt_ref[:]
    gt_cnt = jnp.sum((g_row > g_t).astype(jnp.int32), axis=1, keepdims=True)
    col = lax.broadcasted_iota(jnp.int32, (TB, T), 1)
    row = i * TB + lax.broadcasted_iota(jnp.int32, (TB, T), 0)
    eq_before = (g_row == g_t) & (col < row)
    rank = gt_cnt + jnp.sum(eq_before.astype(jnp.int32), axis=1, keepdims=True)
    gate = jnp.where(rank < k, g_t, 0.0)
    orig = orig_ref[:]
    o_ref[:] = orig + gate * (proc_ref[:] - orig)


def _gate_combine(g_cont, orig, proc, k):
    T = orig.shape[0]
    return pl.pallas_call(
        functools.partial(_gate_combine_kern, k=k, T=T),
        grid=(T // TB,),
        in_specs=[
            pl.BlockSpec((1, T), lambda i: (0, 0)),
            pl.BlockSpec((TB, 1), lambda i: (i, 0)),
            pl.BlockSpec((TB, D), lambda i: (i, 0)),
            pl.BlockSpec((TB, D), lambda i: (i, 0)),
        ],
        out_specs=pl.BlockSpec((TB, D), lambda i: (i, 0)),
        out_shape=jax.ShapeDtypeStruct((T, D), jnp.float32),
    )(g_cont.reshape(1, T), g_cont, orig, proc)


def kernel(hidden_states, beta_ce, beta_cu, ln1, wq, wk, wv, wo, ln2,
           wg, wu, wd, tn_norm, tn_g, tn_u, tn_d):
    B, T, _ = hidden_states.shape
    x = hidden_states.reshape(T, D)

    wqkv = jnp.concatenate([wq, wk, wv], axis=1)
    qkv = _rms_matmul(x, ln1.reshape(1, D), wqkv)
    qkv = qkv.reshape(T, 3, H, DH).transpose(1, 2, 0, 3)
    o = _attention(qkv[0], qkv[1], qkv[2])
    o = o.transpose(1, 0, 2).reshape(T, D)
    h1 = _matmul_add(o, wo, x)

    proc = _mlp_residual(h1, ln2.reshape(1, D), wg, wu, wd)

    prev = jnp.concatenate([jnp.zeros((1, D), jnp.float32), proc[:-1]], axis=0)
    g_cont = _tn_scores(prev, tn_norm.reshape(1, D), tn_g, tn_u, tn_d,
                        proc, x, beta_ce.reshape(1, 1), beta_cu.reshape(1, 1))

    k = max(1, int(T * CAP))
    final = _gate_combine(g_cont, x, proc, k)
    return final.reshape(B, T, D)


# speedup vs baseline: 2.0959x; 2.0959x over previous
import functools

import jax
import jax.numpy as jnp
from jax import lax
from jax.experimental import pallas as pl
from jax.experimental.pallas import tpu as pltpu

D = 1024
F = 2816
H = 16
DH = 64
EPS = 1e-6
CAP = 0.5
TB = 512
FB = 1408


def _qkv_kern(x_ref, s_ref, wq_ref, wk_ref, wv_ref, q_ref, k_ref, v_ref):
    x = x_ref[:]
    var = jnp.mean(jnp.square(x), axis=-1, keepdims=True)
    xn = (x * lax.rsqrt(var + EPS)) * s_ref[:]
    q_ref[:] = jnp.dot(xn, wq_ref[:], preferred_element_type=jnp.float32)
    k_ref[:] = jnp.dot(xn, wk_ref[:], preferred_element_type=jnp.float32)
    v_ref[:] = jnp.dot(xn, wv_ref[:], preferred_element_type=jnp.float32)


def _qkv(x, s, wq, wk, wv):
    T = x.shape[0]
    out = jax.ShapeDtypeStruct((T, D), jnp.float32)
    return pl.pallas_call(
        _qkv_kern,
        grid=(T // TB,),
        in_specs=[
            pl.BlockSpec((TB, D), lambda i: (i, 0)),
            pl.BlockSpec((1, D), lambda i: (0, 0)),
            pl.BlockSpec((D, D), lambda i: (0, 0)),
            pl.BlockSpec((D, D), lambda i: (0, 0)),
            pl.BlockSpec((D, D), lambda i: (0, 0)),
        ],
        out_specs=[pl.BlockSpec((TB, D), lambda i: (i, 0))] * 3,
        out_shape=[out, out, out],
    )(x, s, wq, wk, wv)


def _attn_kern(q_ref, k_ref, v_ref, o_ref):
    qb = pl.program_id(1)
    outs = []
    for off in (0, DH):
        q = q_ref[:, off:off + DH]

        def body(kb, carry, off=off):
            m, l, acc = carry
            start = pl.multiple_of(kb * TB, TB)
            kblk = k_ref[pl.ds(start, TB), off:off + DH]
            vblk = v_ref[pl.ds(start, TB), off:off + DH]
            s = lax.dot_general(q, kblk, (((1,), (1,)), ((), ())),
                                preferred_element_type=jnp.float32) * 0.125
            row = qb * TB + lax.broadcasted_iota(jnp.int32, (TB, TB), 0)
            col = start + lax.broadcasted_iota(jnp.int32, (TB, TB), 1)
            s = jnp.where(row >= col, s, -1e9)
            m_new = jnp.maximum(m, jnp.max(s, axis=1, keepdims=True))
            alpha = jnp.exp(m - m_new)
            p = jnp.exp(s - m_new)
            l_new = l * alpha + jnp.sum(p, axis=1, keepdims=True)
            acc_new = acc * alpha + jnp.dot(p, vblk,
                                            preferred_element_type=jnp.float32)
            return m_new, l_new, acc_new

        m0 = jnp.full((TB, 1), -1e30, jnp.float32)
        l0 = jnp.zeros((TB, 1), jnp.float32)
        acc0 = jnp.zeros((TB, DH), jnp.float32)
        m, l, acc = lax.fori_loop(0, qb + 1, body, (m0, l0, acc0))
        outs.append(acc / l)
    o_ref[:] = jnp.concatenate(outs, axis=1)


def _attention(q, k, v):
    T = q.shape[0]
    return pl.pallas_call(
        _attn_kern,
        grid=(H // 2, T // TB),
        in_specs=[
            pl.BlockSpec((TB, 2 * DH), lambda h, i: (i, h)),
            pl.BlockSpec((T, 2 * DH), lambda h, i: (0, h)),
            pl.BlockSpec((T, 2 * DH), lambda h, i: (0, h)),
        ],
        out_specs=pl.BlockSpec((TB, 2 * DH), lambda h, i: (i, h)),
        out_shape=jax.ShapeDtypeStruct((T, D), jnp.float32),
    )(q, k, v)


def _matmul_add_kern(a_ref, w_ref, r_ref, o_ref):
    o_ref[:] = r_ref[:] + jnp.dot(a_ref[:], w_ref[:],
                                  preferred_element_type=jnp.float32)


def _matmul_add(a, w, r):
    T = a.shape[0]
    N = w.shape[1]
    return pl.pallas_call(
        _matmul_add_kern,
        grid=(T // TB,),
        in_specs=[
            pl.BlockSpec((TB, D), lambda i: (i, 0)),
            pl.BlockSpec((D, N), lambda i: (0, 0)),
            pl.BlockSpec((TB, N), lambda i: (i, 0)),
        ],
        out_specs=pl.BlockSpec((TB, N), lambda i: (i, 0)),
        out_shape=jax.ShapeDtypeStruct((T, N), jnp.float32),
    )(a, w, r)


def _mlp_kern(x_ref, s_ref, wg_ref, wu_ref, wd_ref, o_ref):
    fb = pl.program_id(1)
    x = x_ref[:]
    var = jnp.mean(jnp.square(x), axis=-1, keepdims=True)
    xn = (x * lax.rsqrt(var + EPS)) * s_ref[:]
    g = jnp.dot(xn, wg_ref[:], preferred_element_type=jnp.float32)
    u = jnp.dot(xn, wu_ref[:], preferred_element_type=jnp.float32)
    t = (g * jax.nn.sigmoid(g)) * u
    part = jnp.dot(t, wd_ref[:], preferred_element_type=jnp.float32)

    @pl.when(fb == 0)
    def _():
        o_ref[:] = x + part

    @pl.when(fb != 0)
    def _():
        o_ref[:] = o_ref[:] + part


def _mlp_residual(x, s, wg, wu, wd):
    T = x.shape[0]
    return pl.pallas_call(
        _mlp_kern,
        grid=(T // TB, F // FB),
        in_specs=[
            pl.BlockSpec((TB, D), lambda i, j: (i, 0)),
            pl.BlockSpec((1, D), lambda i, j: (0, 0)),
            pl.BlockSpec((D, FB), lambda i, j: (0, j)),
            pl.BlockSpec((D, FB), lambda i, j: (0, j)),
            pl.BlockSpec((FB, D), lambda i, j: (j, 0)),
        ],
        out_specs=pl.BlockSpec((TB, D), lambda i, j: (i, 0)),
        out_shape=jax.ShapeDtypeStruct((T, D), jnp.float32),
    )(x, s, wg, wu, wd)


def _score_kern(proc_ref, s_ref, wg_ref, wu_ref, wd_ref, orig_ref,
                bce_ref, bcu_ref, g_ref, last_row, prev_s, pred_acc):
    i = pl.program_id(0)
    fb = pl.program_id(1)
    nfb = pl.num_programs(1)

    @pl.when((i == 0) & (fb == 0))
    def _():
        last_row[:] = jnp.zeros((1, D), jnp.float32)

    @pl.when(fb == 0)
    def _():
        proc = proc_ref[:]
        prev_s[:] = jnp.concatenate([last_row[:], proc[:TB - 1]], axis=0)
        last_row[:] = proc[TB - 1:TB]

    x = prev_s[:]
    var = jnp.mean(jnp.square(x), axis=-1, keepdims=True)
    xn = (x * lax.rsqrt(var + EPS)) * s_ref[:]
    g = jnp.dot(xn, wg_ref[:], preferred_element_type=jnp.float32)
    u = jnp.dot(xn, wu_ref[:], preferred_element_type=jnp.float32)
    t = (g * jax.nn.sigmoid(g)) * u
    part = jnp.dot(t, wd_ref[:], preferred_element_type=jnp.float32)

    @pl.when(fb == 0)
    def _():
        pred_acc[:] = part

    @pl.when(fb != 0)
    def _():
        pred_acc[:] = pred_acc[:] + part

    @pl.when(fb == nfb - 1)
    def _():
        pred = pred_acc[:]
        res = proc_ref[:] - orig_ref[:]
        d_st = jnp.sum(res * res, axis=-1, keepdims=True) * (1.0 / D)
        e = res - pred
        d_ch = jnp.sum(e * e, axis=-1, keepdims=True) * (1.0 / D)
        logit = bce_ref[0, 0] * d_st - bcu_ref[0, 0] * d_ch
        g_ref[:] = jax.nn.sigmoid(logit)


def _tn_scores(proc, s, wg, wu, wd, orig, bce, bcu):
    T = proc.shape[0]
    return pl.pallas_call(
        _score_kern,
        grid=(T // TB, F // FB),
        in_specs=[
            pl.BlockSpec((TB, D), lambda i, j: (i, 0)),
            pl.BlockSpec((1, D), lambda i, j: (0, 0)),
            pl.BlockSpec((D, FB), lambda i, j: (0, j)),
            pl.BlockSpec((D, FB), lambda i, j: (0, j)),
            pl.BlockSpec((FB, D), lambda i, j: (j, 0)),
            pl.BlockSpec((TB, D), lambda i, j: (i, 0)),
            pl.BlockSpec((1, 1), lambda i, j: (0, 0)),
            pl.BlockSpec((1, 1), lambda i, j: (0, 0)),
        ],
        out_specs=pl.BlockSpec((TB, 1), lambda i, j: (i, 0)),
        out_shape=jax.ShapeDtypeStruct((T, 1), jnp.float32),
        scratch_shapes=[
            pltpu.VMEM((1, D), jnp.float32),
            pltpu.VMEM((TB, D), jnp.float32),
            pltpu.VMEM((TB, D), jnp.float32),
        ],
    )(proc, s, wg, wu, wd, orig, bce, bcu)


def _gate_combine_kern(grow_ref, gt_ref, orig_ref, proc_ref, o_ref, *, k, T):
    i = pl.program_id(0)
    g_row = grow_ref[:]
    g_t = gt_ref[:]
    gt_cnt = jnp.sum((g_row > g_t).astype(jnp.int32), axis=1, keepdims=True)
    col = lax.broadcasted_iota(jnp.int32, (TB, T), 1)
    row = i * TB + lax.broadcasted_iota(jnp.int32, (TB, T), 0)
    eq_before = (g_row == g_t) & (col < row)
    rank = gt_cnt + jnp.sum(eq_before.astype(jnp.int32), axis=1, keepdims=True)
    gate = jnp.where(rank < k, g_t, 0.0)
    orig = orig_ref[:]
    o_ref[:] = orig + gate * (proc_ref[:] - orig)


def _gate_combine(g_cont, orig, proc, k):
    T = orig.shape[0]
    return pl.pallas_call(
        functools.partial(_gate_combine_kern, k=k, T=T),
        grid=(T // TB,),
        in_specs=[
            pl.BlockSpec((1, T), lambda i: (0, 0)),
            pl.BlockSpec((TB, 1), lambda i: (i, 0)),
            pl.BlockSpec((TB, D), lambda i: (i, 0)),
            pl.BlockSpec((TB, D), lambda i: (i, 0)),
        ],
        out_specs=pl.BlockSpec((TB, D), lambda i: (i, 0)),
        out_shape=jax.ShapeDtypeStruct((T, D), jnp.float32),
    )(g_cont.reshape(1, T), g_cont, orig, proc)


def kernel(hidden_states, beta_ce, beta_cu, ln1, wq, wk, wv, wo, ln2,
           wg, wu, wd, tn_norm, tn_g, tn_u, tn_d):
    B, T, _ = hidden_states.shape
    x = hidden_states.reshape(T, D)

    q, k, v = _qkv(x, ln1.reshape(1, D), wq, wk, wv)
    o = _attention(q, k, v)
    h1 = _matmul_add(o, wo, x)

    proc = _mlp_residual(h1, ln2.reshape(1, D), wg, wu, wd)

    g_cont = _tn_scores(proc, tn_norm.reshape(1, D), tn_g, tn_u, tn_d,
                        x, beta_ce.reshape(1, 1), beta_cu.reshape(1, 1))

    kk = max(1, int(T * CAP))
    final = _gate_combine(g_cont, x, proc, kk)
    return final.reshape(B, T, D)
